# Initial kernel scaffold; baseline (speedup 1.0000x reference)
import jax
import jax.numpy as jnp
from jax.experimental import pallas as pl


def kernel(x, edge_index, batch, mlp_W1, mlp_b1, mlp_g1, mlp_be1, mlp_W2, mlp_b2, bn_g, bn_b, cls_W1, cls_b1, cls_W2, cls_b2):
    raise NotImplementedError("write your pallas kernel here")


# speedup vs baseline: 9.8936x; 9.8936x over previous
import functools

import jax
import jax.numpy as jnp
from jax import lax
from jax.experimental import pallas as pl
from jax.experimental.pallas import tpu as pltpu
from jax.experimental.pallas import tpu_sc as plsc

N = 10000
E = 320000
F_IN = 128
H = 64
HH = H // 2
L = 3
G = 64
C = 2
BN_EPS = 1e-5

NSUB = 16
EPT = E // NSUB
CH = 128
NCH = -(-EPT // CH)
EPT_PAD = NCH * CH
ACC_ROWS = 10240
RPT = ACC_ROWS // NSUB


def _sc_agg_body(pa_hbm, pb_hbm, src_hbm, dst_hbm, z_hbm, out_hbm,
                 src_v, dst_v, rows_v, stage_v, p_sh, acc_sh, sem):
    c = lax.axis_index("c")
    s = lax.axis_index("s")

    pltpu.sync_copy(src_hbm.at[s], src_v)
    pltpu.sync_copy(dst_hbm.at[s], dst_v)
    pltpu.sync_copy(z_hbm, acc_sh.at[pl.ds(s * RPT, RPT)])

    for cc, p_hbm in ((0, pa_hbm), (1, pb_hbm)):
        @pl.when((c == cc) & (s < 15))
        def _():
            pltpu.sync_copy(p_hbm.at[pl.ds(s * RPT, RPT)],
                            p_sh.at[pl.ds(s * RPT, RPT)])

        @pl.when((c == cc) & (s == 15))
        def _():
            pltpu.sync_copy(p_hbm.at[pl.ds(15 * RPT, N - 15 * RPT)],
                            p_sh.at[pl.ds(15 * RPT, N - 15 * RPT)])

    plsc.subcore_barrier()

    def chunk(j, _):
        pltpu.sync_copy(p_sh.at[src_v.at[j]], rows_v)
        pltpu.sync_copy(rows_v, acc_sh.at[dst_v.at[j]], add=True)
        return 0

    lax.fori_loop(0, NCH, chunk, 0)
    plsc.subcore_barrier()

    pltpu.sync_copy(acc_sh.at[pl.ds(s * RPT, RPT)], stage_v)
    pltpu.sync_copy(stage_v, out_hbm.at[c, pl.ds(s * RPT, RPT)])


_sc_agg = functools.partial(
    pl.kernel,
    out_type=jax.ShapeDtypeStruct((2, ACC_ROWS, HH), jnp.float32),
    mesh=plsc.VectorSubcoreMesh(core_axis_name="c", subcore_axis_name="s"),
    compiler_params=pltpu.CompilerParams(use_tc_tiling_on_sc=False),
    scratch_types=[
        pltpu.VMEM((NCH, CH), jnp.int32),
        pltpu.VMEM((NCH, CH), jnp.int32),
        pltpu.VMEM((CH, HH), jnp.float32),
        pltpu.VMEM((RPT, HH), jnp.float32),
        pltpu.VMEM_SHARED((N, HH), jnp.float32),
        pltpu.VMEM_SHARED((ACC_ROWS, HH), jnp.float32),
        pltpu.SemaphoreType.DMA,
    ],
)(_sc_agg_body)


ROW_BLK = 1000
NBLK = N // ROW_BLK


def _proj_body(x_ref, wa_ref, wb_ref, oa_ref, ob_ref):
    x = x_ref[...]
    oa_ref[...] = jnp.dot(x, wa_ref[...], preferred_element_type=jnp.float32)
    ob_ref[...] = jnp.dot(x, wb_ref[...], preferred_element_type=jnp.float32)


def _proj(x, wa, wb):
    fin = x.shape[1]
    return pl.pallas_call(
        _proj_body,
        grid=(NBLK,),
        in_specs=[pl.BlockSpec((ROW_BLK, fin), lambda i: (i, 0)),
                  pl.BlockSpec((fin, HH), lambda i: (0, 0)),
                  pl.BlockSpec((fin, HH), lambda i: (0, 0))],
        out_specs=[pl.BlockSpec((ROW_BLK, HH), lambda i: (i, 0)),
                   pl.BlockSpec((ROW_BLK, HH), lambda i: (i, 0))],
        out_shape=[jax.ShapeDtypeStruct((N, HH), jnp.float32),
                   jax.ShapeDtypeStruct((N, HH), jnp.float32)],
    )(x, wa, wb)


def _tail_body(has_next, pa_ref, pb_ref, a_ref, vec_ref, w2_ref,
               wna_ref, wnb_ref, batch_ref, *out_refs):
    c1 = vec_ref[0, :]
    e1 = vec_ref[1, :]
    c2 = vec_ref[2, :]
    e2 = vec_ref[3, :]
    m = jnp.concatenate([pa_ref[...] + a_ref[0], pb_ref[...] + a_ref[1]],
                        axis=1)
    u = jnp.maximum(m * c1 + e1, 0.0)
    v = jnp.dot(u, w2_ref[...], preferred_element_type=jnp.float32)
    h = jnp.maximum(v * c2 + e2, 0.0)
    if has_next:
        pna_ref, pnb_ref, pool_ref = out_refs
        pna_ref[...] = jnp.dot(h, wna_ref[...],
                               preferred_element_type=jnp.float32)
        pnb_ref[...] = jnp.dot(h, wnb_ref[...],
                               preferred_element_type=jnp.float32)
    else:
        (pool_ref,) = out_refs
    b = batch_ref[0, 0, :]
    onehot = (lax.broadcasted_iota(jnp.int32, (G, ROW_BLK), 0)
              == b[None, :]).astype(jnp.float32)
    seg = jnp.dot(onehot, h, preferred_element_type=jnp.float32)

    @pl.when(pl.program_id(0) == 0)
    def _():
        pool_ref[...] = jnp.zeros_like(pool_ref)

    pool_ref[...] += seg


def _tail(pa, pb, agg, vecs, w2, wna, wnb, batch):
    has_next = wna is not None
    out_shape = [jax.ShapeDtypeStruct((G, H), jnp.float32)]
    out_specs = [pl.BlockSpec((G, H), lambda i: (0, 0))]
    if has_next:
        out_shape = [jax.ShapeDtypeStruct((N, HH), jnp.float32)] * 2 \
            + out_shape
        out_specs = [pl.BlockSpec((ROW_BLK, HH), lambda i: (i, 0))] * 2 \
            + out_specs
    else:
        wna = wnb = jnp.zeros((H, HH), jnp.float32)
    res = pl.pallas_call(
        functools.partial(_tail_body, has_next),
        grid=(NBLK,),
        in_specs=[
            pl.BlockSpec((ROW_BLK, HH), lambda i: (i, 0)),
            pl.BlockSpec((ROW_BLK, HH), lambda i: (i, 0)),
            pl.BlockSpec((2, ROW_BLK, HH), lambda i: (0, i, 0)),
            pl.BlockSpec((8, H), lambda i: (0, 0)),
            pl.BlockSpec((H, H), lambda i: (0, 0)),
            pl.BlockSpec((H, HH), lambda i: (0, 0)),
            pl.BlockSpec((H, HH), lambda i: (0, 0)),
            pl.BlockSpec((1, 1, ROW_BLK), lambda i: (i, 0, 0)),
        ],
        out_specs=out_specs,
        out_shape=out_shape,
    )(pa, pb, agg, vecs, w2, wna, wnb, batch.reshape(NBLK, 1, ROW_BLK))
    if has_next:
        return res[0], res[1], res[2]
    return None, None, res[0]


def _cls_body(p0_ref, p1_ref, p2_ref, wa_ref, wb_ref, wc_ref, b1_ref,
              w2_ref, b2_ref, o_ref):
    z = (jnp.dot(p0_ref[...], wa_ref[...], preferred_element_type=jnp.float32)
         + jnp.dot(p1_ref[...], wb_ref[...],
                   preferred_element_type=jnp.float32)
         + jnp.dot(p2_ref[...], wc_ref[...],
                   preferred_element_type=jnp.float32))
    z = jnp.maximum(z + b1_ref[...], 0.0)
    o_ref[...] = jnp.dot(z, w2_ref[...],
                         preferred_element_type=jnp.float32) + b2_ref[...]


def _classifier(p0, p1, p2, wa, wb, wc, b1, w2, b2):
    return pl.pallas_call(
        _cls_body,
        out_shape=jax.ShapeDtypeStruct((G, C), jnp.float32),
    )(p0, p1, p2, wa, wb, wc, b1, w2, b2)


def kernel(x, edge_index, batch, mlp_W1, mlp_b1, mlp_g1, mlp_be1, mlp_W2,
           mlp_b2, bn_g, bn_b, cls_W1, cls_b1, cls_W2, cls_b2):
    src = edge_index[0]
    dst = edge_index[1]
    pad = EPT_PAD - EPT
    src_t = jnp.pad(src.reshape(NSUB, EPT), ((0, 0), (0, pad))) \
        .reshape(NSUB, NCH, CH)
    dst_t = jnp.pad(dst.reshape(NSUB, EPT), ((0, 0), (0, pad)),
                    constant_values=N).reshape(NSUB, NCH, CH)
    zeros = jnp.zeros((RPT, HH), jnp.float32)

    s0 = 1.0 / jnp.sqrt(jnp.float32(1.0 + BN_EPS))
    pooled = []
    pa, pb = _proj(x, mlp_W1[0][:, :HH], mlp_W1[0][:, HH:])
    for i in range(L):
        c1 = mlp_g1[i] * s0
        e1 = mlp_b1[i] * c1 + mlp_be1[i]
        c2 = bn_g[i] * s0
        e2 = mlp_b2[i] * c2 + bn_b[i]
        vecs = jnp.zeros((8, H), jnp.float32)
        vecs = vecs.at[0].set(c1).at[1].set(e1).at[2].set(c2).at[3].set(e2)
        agg = _sc_agg(pa, pb, src_t, dst_t, zeros)
        if i + 1 < L:
            wna, wnb = mlp_W1[i + 1][:, :HH], mlp_W1[i + 1][:, HH:]
        else:
            wna = wnb = None
        pa, pb, pool = _tail(pa, pb, agg, vecs, mlp_W2[i], wna, wnb, batch)
        pooled.append(pool)

    return _classifier(pooled[0], pooled[1], pooled[2],
                       cls_W1[0:H], cls_W1[H:2 * H], cls_W1[2 * H:3 * H],
                       cls_b1.reshape(1, H), cls_W2, cls_b2.reshape(1, C))
